# Initial kernel scaffold; baseline (speedup 1.0000x reference)
import functools

import jax
import jax.numpy as jnp
from jax import lax
from jax.experimental import pallas as pl
from jax.experimental.pallas import tpu as pltpu
from jax.experimental.pallas import tpu_sc as plsc

N = 10000
E = 160000
D = 256
DH = 128

NC = 2
NS = 16
LANES = 16

NPAD = 10240
ROWS_PER_TILE = NPAD // NS
ROW_CHUNKS = ROWS_PER_TILE // 128
EC = 128
CHUNKS_PER_TILE = 79
EPT = CHUNKS_PER_TILE * EC
EPAD = NS * EPT
PAD_NODE = N

_MESH = plsc.VectorSubcoreMesh(core_axis_name="c", subcore_axis_name="s")


def _fill2d(ref, nrows, ncolchunks, val):
    v = jnp.full((LANES,), val, dtype=jnp.float32)

    def body(i, carry):
        for cc in range(ncolchunks):
            ref[i, pl.ds(cc * LANES, LANES)] = v
        return carry

    lax.fori_loop(0, nrows, body, 0)


@functools.partial(
    pl.kernel,
    out_type=(
        jax.ShapeDtypeStruct((NPAD, LANES), jnp.float32),
        jax.ShapeDtypeStruct((NPAD, LANES), jnp.float32),
    ),
    mesh=_MESH,
    scratch_types=[
        pltpu.VMEM((CHUNKS_PER_TILE, EC), jnp.int32),
        pltpu.VMEM((EC, LANES), jnp.float32),
        pltpu.VMEM_SHARED((NPAD, LANES), jnp.float32),
    ],
)
def _deg_kernel(src2d, dst2d, deg_out, deg_in, idx_v, buf_v, shared):
    c = lax.axis_index("c")
    s = lax.axis_index("s")

    _fill2d(buf_v, EC, 1, 0.0)
    for t in range(ROW_CHUNKS):
        pltpu.sync_copy(buf_v, shared.at[pl.ds(s * ROWS_PER_TILE + t * 128, 128)])

    @pl.when(c == 0)
    def _():
        pltpu.sync_copy(src2d.at[pl.ds(s * CHUNKS_PER_TILE, CHUNKS_PER_TILE)], idx_v)

    @pl.when(c == 1)
    def _():
        pltpu.sync_copy(dst2d.at[pl.ds(s * CHUNKS_PER_TILE, CHUNKS_PER_TILE)], idx_v)

    _fill2d(buf_v, EC, 1, 1.0)
    plsc.subcore_barrier()

    def body(j, carry):
        pltpu.sync_copy(buf_v, shared.at[idx_v.at[j]], add=True)
        return carry

    lax.fori_loop(0, CHUNKS_PER_TILE, body, 0)
    plsc.subcore_barrier()

    for t in range(ROW_CHUNKS):
        r0 = s * ROWS_PER_TILE + t * 128
        pltpu.sync_copy(shared.at[pl.ds(r0, 128)], buf_v)

        @pl.when(c == 0)
        def _():
            pltpu.sync_copy(buf_v, deg_out.at[pl.ds(r0, 128)])

        @pl.when(c == 1)
        def _():
            pltpu.sync_copy(buf_v, deg_in.at[pl.ds(r0, 128)])


@functools.partial(
    pl.kernel,
    out_type=(
        jax.ShapeDtypeStruct((NPAD, DH), jnp.float32),
        jax.ShapeDtypeStruct((NPAD, DH), jnp.float32),
    ),
    mesh=_MESH,
    scratch_types=[
        pltpu.VMEM((CHUNKS_PER_TILE, EC), jnp.int32),
        pltpu.VMEM((CHUNKS_PER_TILE, EC), jnp.int32),
        pltpu.VMEM((EC, DH), jnp.float32),
        pltpu.VMEM_SHARED((NPAD, DH), jnp.float32),
        pltpu.SemaphoreType.DMA,
    ],
)
def _agg_kernel(hs_a, hs_b, src2d, dst2d, agg_a, agg_b,
                idxs_v, idxd_v, rows_v, shared, sem):
    c = lax.axis_index("c")
    s = lax.axis_index("s")

    _fill2d(rows_v, EC, DH // LANES, 0.0)
    for t in range(ROW_CHUNKS):
        pltpu.sync_copy(rows_v, shared.at[pl.ds(s * ROWS_PER_TILE + t * 128, 128)])

    pltpu.sync_copy(src2d.at[pl.ds(s * CHUNKS_PER_TILE, CHUNKS_PER_TILE)], idxs_v)
    pltpu.sync_copy(dst2d.at[pl.ds(s * CHUNKS_PER_TILE, CHUNKS_PER_TILE)], idxd_v)
    plsc.subcore_barrier()

    def body(j, carry):
        @pl.when(c == 0)
        def _():
            pltpu.async_copy(hs_a.at[idxs_v.at[j]], rows_v, sem).wait()

        @pl.when(c == 1)
        def _():
            pltpu.async_copy(hs_b.at[idxs_v.at[j]], rows_v, sem).wait()

        pltpu.sync_copy(rows_v, shared.at[idxd_v.at[j]], add=True)
        return carry

    lax.fori_loop(0, CHUNKS_PER_TILE, body, 0)
    plsc.subcore_barrier()

    for t in range(ROW_CHUNKS):
        r0 = s * ROWS_PER_TILE + t * 128
        pltpu.sync_copy(shared.at[pl.ds(r0, 128)], rows_v)

        @pl.when(c == 0)
        def _():
            pltpu.sync_copy(rows_v, agg_a.at[pl.ds(r0, 128)])

        @pl.when(c == 1)
        def _():
            pltpu.sync_copy(rows_v, agg_b.at[pl.ds(r0, 128)])


def _pre_body(do_ref, di_ref, x_ref, ns_ref, nd_ref, ha_ref, hb_ref):
    do = jnp.sum(do_ref[...], axis=1, keepdims=True)
    di = jnp.sum(di_ref[...], axis=1, keepdims=True)
    ns = jnp.where(do > 0, lax.rsqrt(do), 0.0)
    nd = jnp.where(di > 0, lax.rsqrt(di), 0.0)
    ns_ref[...] = jnp.broadcast_to(ns, (128, DH))
    nd_ref[...] = jnp.broadcast_to(nd, (128, DH))
    hs = x_ref[...] * ns
    ha_ref[...] = hs[:, :DH]
    hb_ref[...] = hs[:, DH:]


_pre_call = pl.pallas_call(
    _pre_body,
    grid=(NPAD // 128,),
    in_specs=[
        pl.BlockSpec((128, LANES), lambda i: (i, 0)),
        pl.BlockSpec((128, LANES), lambda i: (i, 0)),
        pl.BlockSpec((128, D), lambda i: (i, 0)),
    ],
    out_specs=[
        pl.BlockSpec((128, DH), lambda i: (i, 0)),
        pl.BlockSpec((128, DH), lambda i: (i, 0)),
        pl.BlockSpec((128, DH), lambda i: (i, 0)),
        pl.BlockSpec((128, DH), lambda i: (i, 0)),
    ],
    out_shape=[
        jax.ShapeDtypeStruct((NPAD, DH), jnp.float32),
        jax.ShapeDtypeStruct((NPAD, DH), jnp.float32),
        jax.ShapeDtypeStruct((NPAD, DH), jnp.float32),
        jax.ShapeDtypeStruct((NPAD, DH), jnp.float32),
    ],
)


def _layer_body(aa_ref, ab_ref, nd_ref, ns_ref, w_ref, b_ref, ha_ref, hb_ref):
    w = w_ref[...]
    t = jnp.dot(aa_ref[...], w[:DH, :], preferred_element_type=jnp.float32)
    t += jnp.dot(ab_ref[...], w[DH:, :], preferred_element_type=jnp.float32)
    t = t * nd_ref[:, :1]
    h = jnp.maximum(t + b_ref[...], 0.0)
    hs = h * ns_ref[:, :1]
    ha_ref[...] = hs[:, :DH]
    hb_ref[...] = hs[:, DH:]


_layer_call = pl.pallas_call(
    _layer_body,
    grid=(NPAD // 128,),
    in_specs=[
        pl.BlockSpec((128, DH), lambda i: (i, 0)),
        pl.BlockSpec((128, DH), lambda i: (i, 0)),
        pl.BlockSpec((128, DH), lambda i: (i, 0)),
        pl.BlockSpec((128, DH), lambda i: (i, 0)),
        pl.BlockSpec((D, D), lambda i: (0, 0)),
        pl.BlockSpec((1, D), lambda i: (0, 0)),
    ],
    out_specs=[
        pl.BlockSpec((128, DH), lambda i: (i, 0)),
        pl.BlockSpec((128, DH), lambda i: (i, 0)),
    ],
    out_shape=[
        jax.ShapeDtypeStruct((NPAD, DH), jnp.float32),
        jax.ShapeDtypeStruct((NPAD, DH), jnp.float32),
    ],
)


def _final_body(aa_ref, ab_ref, nd_ref, w_ref, b_ref, h_ref, hc_ref):
    w = w_ref[...]
    t = jnp.dot(aa_ref[...], w[:DH, :], preferred_element_type=jnp.float32)
    t += jnp.dot(ab_ref[...], w[DH:, :], preferred_element_type=jnp.float32)
    t = t * nd_ref[:, :1]
    h = jnp.maximum(t + b_ref[...], 0.0)
    h_ref[...] = h
    hc_ref[...] = jnp.where(h >= 0.5, jnp.float32(1.0), jnp.float32(0.0))


_final_call = pl.pallas_call(
    _final_body,
    grid=(NPAD // 128,),
    in_specs=[
        pl.BlockSpec((128, DH), lambda i: (i, 0)),
        pl.BlockSpec((128, DH), lambda i: (i, 0)),
        pl.BlockSpec((128, DH), lambda i: (i, 0)),
        pl.BlockSpec((D, D), lambda i: (0, 0)),
        pl.BlockSpec((1, D), lambda i: (0, 0)),
    ],
    out_specs=[
        pl.BlockSpec((128, D), lambda i: (i, 0)),
        pl.BlockSpec((128, D), lambda i: (i, 0)),
    ],
    out_shape=[
        jax.ShapeDtypeStruct((NPAD, D), jnp.float32),
        jax.ShapeDtypeStruct((NPAD, D), jnp.float32),
    ],
)


def kernel(x, edge_index, W1, W2, W3, W4, W5, b1, b2, b3, b4, b5):
    src = edge_index[0]
    dst = edge_index[1]
    pad = jnp.full((EPAD - E,), PAD_NODE, dtype=jnp.int32)
    src2d = jnp.concatenate([src, pad]).reshape(NS * CHUNKS_PER_TILE, EC)
    dst2d = jnp.concatenate([dst, pad]).reshape(NS * CHUNKS_PER_TILE, EC)
    xp = jnp.pad(x, ((0, NPAD - N), (0, 0)))

    deg_out16, deg_in16 = _deg_kernel(src2d, dst2d)
    ns, nd, ha, hb = _pre_call(deg_out16, deg_in16, xp)

    for W, b in ((W1, b1), (W2, b2), (W3, b3), (W4, b4)):
        aa, ab = _agg_kernel(ha, hb, src2d, dst2d)
        ha, hb = _layer_call(aa, ab, nd, ns, W, b.reshape(1, D))

    aa, ab = _agg_kernel(ha, hb, src2d, dst2d)
    h, hc = _final_call(aa, ab, nd, W5, b5.reshape(1, D))
    return h[:N], hc[:N]


# speedup vs baseline: 2.6425x; 2.6425x over previous
import functools

import jax
import jax.numpy as jnp
from jax import lax
from jax.experimental import pallas as pl
from jax.experimental.pallas import tpu as pltpu
from jax.experimental.pallas import tpu_sc as plsc

N = 10000
E = 160000
D = 256
DH = 128

NC = 2
NS = 16
LANES = 16

NPAD = 10240
ROWS_PER_TILE = NPAD // NS
ROW_CHUNKS = ROWS_PER_TILE // 128
EC = 128
CHUNKS_PER_TILE = 80
EPT = CHUNKS_PER_TILE * EC
EPAD = NS * EPT
PAD_NODE = N

_MESH = plsc.VectorSubcoreMesh(core_axis_name="c", subcore_axis_name="s")


def _fill2d(ref, nrows, ncolchunks, val):
    v = jnp.full((LANES,), val, dtype=jnp.float32)

    def body(i, carry):
        for cc in range(ncolchunks):
            ref[i, pl.ds(cc * LANES, LANES)] = v
        return carry

    lax.fori_loop(0, nrows, body, 0)


DEG_CPT = NS * CHUNKS_PER_TILE // (NC * NS)


def _fill_lane(ref, lane):
    i16 = lax.iota(jnp.int32, LANES)

    def body(i, carry):
        for cc in range(DH // LANES):
            v = jnp.where(i16 + cc * LANES == lane, jnp.float32(1.0),
                          jnp.float32(0.0))
            ref[i, pl.ds(cc * LANES, LANES)] = v
        return carry

    lax.fori_loop(0, EC, body, 0)


@functools.partial(
    pl.kernel,
    out_type=jax.ShapeDtypeStruct((NC, NPAD, DH), jnp.float32),
    mesh=_MESH,
    scratch_types=[
        pltpu.VMEM((DEG_CPT, EC), jnp.int32),
        pltpu.VMEM((DEG_CPT, EC), jnp.int32),
        pltpu.VMEM((EC, DH), jnp.float32),
        pltpu.VMEM((EC, DH), jnp.float32),
        pltpu.VMEM_SHARED((NPAD, DH), jnp.float32),
    ],
)
def _deg_kernel(edges3, deg3, idxs_v, idxd_v, bufa_v, bufb_v, shared):
    c = lax.axis_index("c")
    s = lax.axis_index("s")

    _fill2d(bufa_v, EC, DH // LANES, 0.0)
    for t in range(ROW_CHUNKS):
        pltpu.sync_copy(bufa_v, shared.at[pl.ds(s * ROWS_PER_TILE + t * 128, 128)])

    base = (c * NS + s) * DEG_CPT
    pltpu.sync_copy(edges3.at[0, pl.ds(base, DEG_CPT)], idxs_v)
    pltpu.sync_copy(edges3.at[1, pl.ds(base, DEG_CPT)], idxd_v)
    _fill_lane(bufa_v, 0)
    _fill_lane(bufb_v, 1)
    plsc.subcore_barrier()

    def body(j, carry):
        pltpu.sync_copy(bufa_v, shared.at[idxs_v.at[j]], add=True)
        pltpu.sync_copy(bufb_v, shared.at[idxd_v.at[j]], add=True)
        return carry

    lax.fori_loop(0, DEG_CPT, body, 0)
    plsc.subcore_barrier()

    for t in range(ROW_CHUNKS):
        r0 = s * ROWS_PER_TILE + t * 128
        pltpu.sync_copy(shared.at[pl.ds(r0, 128)], bufa_v)
        pltpu.sync_copy(bufa_v, deg3.at[c, pl.ds(r0, 128)])


@functools.partial(
    pl.kernel,
    out_type=jax.ShapeDtypeStruct((NC, NPAD, DH), jnp.float32),
    mesh=_MESH,
    scratch_types=[
        pltpu.VMEM((CHUNKS_PER_TILE, EC), jnp.int32),
        pltpu.VMEM((CHUNKS_PER_TILE, EC), jnp.int32),
        pltpu.VMEM((EC, DH), jnp.float32),
        pltpu.VMEM_SHARED((NPAD, DH), jnp.float32),
        pltpu.SemaphoreType.DMA,
    ],
)
def _agg_kernel(hs3, edges3, agg3, idxs_v, idxd_v, rows_v, shared, sem):
    c = lax.axis_index("c")
    s = lax.axis_index("s")

    _fill2d(rows_v, EC, DH // LANES, 0.0)
    for t in range(ROW_CHUNKS):
        pltpu.sync_copy(rows_v, shared.at[pl.ds(s * ROWS_PER_TILE + t * 128, 128)])

    pltpu.sync_copy(edges3.at[0, pl.ds(s * CHUNKS_PER_TILE, CHUNKS_PER_TILE)], idxs_v)
    pltpu.sync_copy(edges3.at[1, pl.ds(s * CHUNKS_PER_TILE, CHUNKS_PER_TILE)], idxd_v)
    plsc.subcore_barrier()

    def body(j, carry):
        pltpu.async_copy(hs3.at[c].at[idxs_v.at[j]], rows_v, sem).wait()
        pltpu.sync_copy(rows_v, shared.at[idxd_v.at[j]], add=True)
        return carry

    lax.fori_loop(0, CHUNKS_PER_TILE, body, 0)
    plsc.subcore_barrier()

    for t in range(ROW_CHUNKS):
        r0 = s * ROWS_PER_TILE + t * 128
        pltpu.sync_copy(shared.at[pl.ds(r0, 128)], rows_v)
        pltpu.sync_copy(rows_v, agg3.at[c, pl.ds(r0, 128)])


def _pre_body(deg_ref, x_ref, ns_ref, nd_ref, hs_ref):
    do = deg_ref[0, :, 0:1] + deg_ref[1, :, 0:1]
    di = deg_ref[0, :, 1:2] + deg_ref[1, :, 1:2]
    ns = jnp.where(do > 0, lax.rsqrt(do), 0.0)
    nd = jnp.where(di > 0, lax.rsqrt(di), 0.0)
    ns_ref[...] = jnp.broadcast_to(ns, (128, DH))
    nd_ref[...] = jnp.broadcast_to(nd, (128, DH))
    hs = x_ref[...] * ns
    hs_ref[0] = hs[:, :DH]
    hs_ref[1] = hs[:, DH:]


_pre_call = pl.pallas_call(
    _pre_body,
    grid=(NPAD // 128,),
    in_specs=[
        pl.BlockSpec((NC, 128, DH), lambda i: (0, i, 0)),
        pl.BlockSpec((128, D), lambda i: (i, 0)),
    ],
    out_specs=[
        pl.BlockSpec((128, DH), lambda i: (i, 0)),
        pl.BlockSpec((128, DH), lambda i: (i, 0)),
        pl.BlockSpec((NC, 128, DH), lambda i: (0, i, 0)),
    ],
    out_shape=[
        jax.ShapeDtypeStruct((NPAD, DH), jnp.float32),
        jax.ShapeDtypeStruct((NPAD, DH), jnp.float32),
        jax.ShapeDtypeStruct((NC, NPAD, DH), jnp.float32),
    ],
)


def _layer_body(agg_ref, nd_ref, ns_ref, w_ref, b_ref, hs_ref):
    w = w_ref[...]
    t = jnp.dot(agg_ref[0], w[:DH, :], preferred_element_type=jnp.float32)
    t += jnp.dot(agg_ref[1], w[DH:, :], preferred_element_type=jnp.float32)
    t = t * nd_ref[:, :1]
    h = jnp.maximum(t + b_ref[...], 0.0)
    hs = h * ns_ref[:, :1]
    hs_ref[0] = hs[:, :DH]
    hs_ref[1] = hs[:, DH:]


_layer_call = pl.pallas_call(
    _layer_body,
    grid=(NPAD // 128,),
    in_specs=[
        pl.BlockSpec((NC, 128, DH), lambda i: (0, i, 0)),
        pl.BlockSpec((128, DH), lambda i: (i, 0)),
        pl.BlockSpec((128, DH), lambda i: (i, 0)),
        pl.BlockSpec((D, D), lambda i: (0, 0)),
        pl.BlockSpec((1, D), lambda i: (0, 0)),
    ],
    out_specs=[
        pl.BlockSpec((NC, 128, DH), lambda i: (0, i, 0)),
    ],
    out_shape=[
        jax.ShapeDtypeStruct((NC, NPAD, DH), jnp.float32),
    ],
)


def _final_body(agg_ref, nd_ref, w_ref, b_ref, h_ref, hc_ref):
    w = w_ref[...]
    t = jnp.dot(agg_ref[0], w[:DH, :], preferred_element_type=jnp.float32)
    t += jnp.dot(agg_ref[1], w[DH:, :], preferred_element_type=jnp.float32)
    t = t * nd_ref[:, :1]
    h = jnp.maximum(t + b_ref[...], 0.0)
    h_ref[...] = h
    hc_ref[...] = jnp.where(h >= 0.5, jnp.float32(1.0), jnp.float32(0.0))


_final_call = pl.pallas_call(
    _final_body,
    grid=(NPAD // 128,),
    in_specs=[
        pl.BlockSpec((NC, 128, DH), lambda i: (0, i, 0)),
        pl.BlockSpec((128, DH), lambda i: (i, 0)),
        pl.BlockSpec((D, D), lambda i: (0, 0)),
        pl.BlockSpec((1, D), lambda i: (0, 0)),
    ],
    out_specs=[
        pl.BlockSpec((128, D), lambda i: (i, 0)),
        pl.BlockSpec((128, D), lambda i: (i, 0)),
    ],
    out_shape=[
        jax.ShapeDtypeStruct((NPAD, D), jnp.float32),
        jax.ShapeDtypeStruct((NPAD, D), jnp.float32),
    ],
)


def kernel(x, edge_index, W1, W2, W3, W4, W5, b1, b2, b3, b4, b5):
    epad = jnp.full((2, EPAD - E), PAD_NODE, dtype=jnp.int32)
    edges3 = jnp.concatenate([edge_index, epad], axis=1).reshape(
        2, NS * CHUNKS_PER_TILE, EC)
    xp = jnp.pad(x, ((0, NPAD - N), (0, 0)))

    deg3 = _deg_kernel(edges3)
    ns, nd, hs3 = _pre_call(deg3, xp)

    for W, b in ((W1, b1), (W2, b2), (W3, b3), (W4, b4)):
        agg3 = _agg_kernel(hs3, edges3)
        (hs3,) = _layer_call(agg3, nd, ns, W, b.reshape(1, D))

    agg3 = _agg_kernel(hs3, edges3)
    h, hc = _final_call(agg3, nd, W5, b5.reshape(1, D))
    return h[:N], hc[:N]


# speedup vs baseline: 3.1224x; 1.1816x over previous
import functools

import jax
import jax.numpy as jnp
from jax import lax
from jax.experimental import pallas as pl
from jax.experimental.pallas import tpu as pltpu
from jax.experimental.pallas import tpu_sc as plsc

N = 10000
E = 160000
D = 256
DH = 128

NC = 2
NS = 16
LANES = 16

NPAD = 10240
ROWS_PER_TILE = NPAD // NS
ROW_CHUNKS = ROWS_PER_TILE // 128
EC = 128
CHUNKS_PER_TILE = 80
EPT = CHUNKS_PER_TILE * EC
EPAD = NS * EPT
PAD_NODE = N

_MESH = plsc.VectorSubcoreMesh(core_axis_name="c", subcore_axis_name="s")


def _fill2d(ref, nrows, ncolchunks, val):
    v = jnp.full((LANES,), val, dtype=jnp.float32)

    def body(i, carry):
        for cc in range(ncolchunks):
            ref[i, pl.ds(cc * LANES, LANES)] = v
        return carry

    lax.fori_loop(0, nrows, body, 0)


DEG_CPT = NS * CHUNKS_PER_TILE // (NC * NS)


def _fill_lane(ref, lane):
    i16 = lax.iota(jnp.int32, LANES)

    def body(i, carry):
        for cc in range(DH // LANES):
            v = jnp.where(i16 + cc * LANES == lane, jnp.float32(1.0),
                          jnp.float32(0.0))
            ref[i, pl.ds(cc * LANES, LANES)] = v
        return carry

    lax.fori_loop(0, EC, body, 0)


@functools.partial(
    pl.kernel,
    out_type=jax.ShapeDtypeStruct((NC, NPAD, DH), jnp.float32),
    mesh=_MESH,
    scratch_types=[
        pltpu.VMEM((DEG_CPT, EC), jnp.int32),
        pltpu.VMEM((DEG_CPT, EC), jnp.int32),
        pltpu.VMEM((EC, DH), jnp.float32),
        pltpu.VMEM((EC, DH), jnp.float32),
        pltpu.VMEM_SHARED((NPAD, DH), jnp.float32),
    ],
)
def _deg_kernel(edges3, deg3, idxs_v, idxd_v, bufa_v, bufb_v, shared):
    c = lax.axis_index("c")
    s = lax.axis_index("s")

    _fill2d(bufa_v, EC, DH // LANES, 0.0)
    for t in range(ROW_CHUNKS):
        pltpu.sync_copy(bufa_v, shared.at[pl.ds(s * ROWS_PER_TILE + t * 128, 128)])

    base = (c * NS + s) * DEG_CPT
    pltpu.sync_copy(edges3.at[0, pl.ds(base, DEG_CPT)], idxs_v)
    pltpu.sync_copy(edges3.at[1, pl.ds(base, DEG_CPT)], idxd_v)
    _fill_lane(bufa_v, 0)
    _fill_lane(bufb_v, 1)
    plsc.subcore_barrier()

    def body(j, carry):
        pltpu.sync_copy(bufa_v, shared.at[idxs_v.at[j]], add=True)
        pltpu.sync_copy(bufb_v, shared.at[idxd_v.at[j]], add=True)
        return carry

    lax.fori_loop(0, DEG_CPT, body, 0)
    plsc.subcore_barrier()

    for t in range(ROW_CHUNKS):
        r0 = s * ROWS_PER_TILE + t * 128
        pltpu.sync_copy(shared.at[pl.ds(r0, 128)], bufa_v)
        pltpu.sync_copy(bufa_v, deg3.at[c, pl.ds(r0, 128)])


NBUF = 2
HALF = CHUNKS_PER_TILE // 2
HGROUPS = HALF // NBUF


@functools.partial(
    pl.kernel,
    out_type=jax.ShapeDtypeStruct((NC, NPAD, DH), jnp.float32),
    mesh=_MESH,
    scratch_types=[
        pltpu.VMEM((HALF, EC), jnp.int32),
        pltpu.VMEM((HALF, EC), jnp.int32),
        pltpu.VMEM((NBUF * EC, DH), jnp.float32),
        pltpu.VMEM_SHARED((NPAD, DH), jnp.float32),
        pltpu.SemaphoreType.DMA,
        pltpu.SemaphoreType.DMA,
    ],
)
def _agg_kernel(hs3, edges3, agg3, idxs_v, idxd_v, rows_v, shared, sem0, sem1):
    c = lax.axis_index("c")
    s = lax.axis_index("s")
    sems = (sem0, sem1)

    def buf(b):
        return rows_v.at[pl.ds(b * EC, EC)]

    _fill2d(rows_v, EC, DH // LANES, 0.0)
    for t in range(ROW_CHUNKS):
        pltpu.sync_copy(buf(0), shared.at[pl.ds(s * ROWS_PER_TILE + t * 128, 128)])
    plsc.subcore_barrier()

    for h in range(2):
        base = s * CHUNKS_PER_TILE + h * HALF
        pltpu.sync_copy(edges3.at[0, pl.ds(base, HALF)], idxs_v)
        pltpu.sync_copy(edges3.at[1, pl.ds(base, HALF)], idxd_v)

        for b in range(NBUF):
            pltpu.async_copy(hs3.at[c].at[idxs_v.at[b]], buf(b), sems[b])

        def body(g, carry):
            j0 = g * NBUF
            for b in range(NBUF):
                j = j0 + b
                pltpu.make_async_copy(hs3.at[c].at[idxs_v.at[j]], buf(b), sems[b]).wait()
                pltpu.sync_copy(buf(b), shared.at[idxd_v.at[j]], add=True)
                pltpu.async_copy(hs3.at[c].at[idxs_v.at[j + NBUF]], buf(b), sems[b])
            return carry

        lax.fori_loop(0, HGROUPS - 1, body, 0)
        j0 = (HGROUPS - 1) * NBUF
        for b in range(NBUF):
            pltpu.make_async_copy(hs3.at[c].at[idxs_v.at[j0 + b]], buf(b), sems[b]).wait()
            pltpu.sync_copy(buf(b), shared.at[idxd_v.at[j0 + b]], add=True)

    plsc.subcore_barrier()

    for t in range(ROW_CHUNKS):
        r0 = s * ROWS_PER_TILE + t * 128
        pltpu.sync_copy(shared.at[pl.ds(r0, 128)], buf(0))
        pltpu.sync_copy(buf(0), agg3.at[c, pl.ds(r0, 128)])


def _pre_body(deg_ref, x_ref, ns_ref, nd_ref, hs_ref):
    do = deg_ref[0, :, 0:1] + deg_ref[1, :, 0:1]
    di = deg_ref[0, :, 1:2] + deg_ref[1, :, 1:2]
    ns = jnp.where(do > 0, lax.rsqrt(do), 0.0)
    nd = jnp.where(di > 0, lax.rsqrt(di), 0.0)
    ns_ref[...] = jnp.broadcast_to(ns, (128, DH))
    nd_ref[...] = jnp.broadcast_to(nd, (128, DH))
    hs = x_ref[...] * ns
    hs_ref[0] = hs[:, :DH]
    hs_ref[1] = hs[:, DH:]


_pre_call = pl.pallas_call(
    _pre_body,
    grid=(NPAD // 128,),
    in_specs=[
        pl.BlockSpec((NC, 128, DH), lambda i: (0, i, 0)),
        pl.BlockSpec((128, D), lambda i: (i, 0)),
    ],
    out_specs=[
        pl.BlockSpec((128, DH), lambda i: (i, 0)),
        pl.BlockSpec((128, DH), lambda i: (i, 0)),
        pl.BlockSpec((NC, 128, DH), lambda i: (0, i, 0)),
    ],
    out_shape=[
        jax.ShapeDtypeStruct((NPAD, DH), jnp.float32),
        jax.ShapeDtypeStruct((NPAD, DH), jnp.float32),
        jax.ShapeDtypeStruct((NC, NPAD, DH), jnp.float32),
    ],
)


def _layer_body(agg_ref, nd_ref, ns_ref, w_ref, b_ref, hs_ref):
    w = w_ref[...]
    t = jnp.dot(agg_ref[0], w[:DH, :], preferred_element_type=jnp.float32)
    t += jnp.dot(agg_ref[1], w[DH:, :], preferred_element_type=jnp.float32)
    t = t * nd_ref[:, :1]
    h = jnp.maximum(t + b_ref[...], 0.0)
    hs = h * ns_ref[:, :1]
    hs_ref[0] = hs[:, :DH]
    hs_ref[1] = hs[:, DH:]


_layer_call = pl.pallas_call(
    _layer_body,
    grid=(NPAD // 128,),
    in_specs=[
        pl.BlockSpec((NC, 128, DH), lambda i: (0, i, 0)),
        pl.BlockSpec((128, DH), lambda i: (i, 0)),
        pl.BlockSpec((128, DH), lambda i: (i, 0)),
        pl.BlockSpec((D, D), lambda i: (0, 0)),
        pl.BlockSpec((1, D), lambda i: (0, 0)),
    ],
    out_specs=[
        pl.BlockSpec((NC, 128, DH), lambda i: (0, i, 0)),
    ],
    out_shape=[
        jax.ShapeDtypeStruct((NC, NPAD, DH), jnp.float32),
    ],
)


def _final_body(agg_ref, nd_ref, w_ref, b_ref, h_ref, hc_ref):
    w = w_ref[...]
    t = jnp.dot(agg_ref[0], w[:DH, :], preferred_element_type=jnp.float32)
    t += jnp.dot(agg_ref[1], w[DH:, :], preferred_element_type=jnp.float32)
    t = t * nd_ref[:, :1]
    h = jnp.maximum(t + b_ref[...], 0.0)
    h_ref[...] = h
    hc_ref[...] = jnp.where(h >= 0.5, jnp.float32(1.0), jnp.float32(0.0))


_final_call = pl.pallas_call(
    _final_body,
    grid=(NPAD // 128,),
    in_specs=[
        pl.BlockSpec((NC, 128, DH), lambda i: (0, i, 0)),
        pl.BlockSpec((128, DH), lambda i: (i, 0)),
        pl.BlockSpec((D, D), lambda i: (0, 0)),
        pl.BlockSpec((1, D), lambda i: (0, 0)),
    ],
    out_specs=[
        pl.BlockSpec((128, D), lambda i: (i, 0)),
        pl.BlockSpec((128, D), lambda i: (i, 0)),
    ],
    out_shape=[
        jax.ShapeDtypeStruct((NPAD, D), jnp.float32),
        jax.ShapeDtypeStruct((NPAD, D), jnp.float32),
    ],
)


def kernel(x, edge_index, W1, W2, W3, W4, W5, b1, b2, b3, b4, b5):
    epad = jnp.full((2, EPAD - E), PAD_NODE, dtype=jnp.int32)
    edges3 = jnp.concatenate([edge_index, epad], axis=1).reshape(
        2, NS * CHUNKS_PER_TILE, EC)
    xp = jnp.pad(x, ((0, NPAD - N), (0, 0)))

    deg3 = _deg_kernel(edges3)
    ns, nd, hs3 = _pre_call(deg3, xp)

    for W, b in ((W1, b1), (W2, b2), (W3, b3), (W4, b4)):
        agg3 = _agg_kernel(hs3, edges3)
        (hs3,) = _layer_call(agg3, nd, ns, W, b.reshape(1, D))

    agg3 = _agg_kernel(hs3, edges3)
    h, hc = _final_call(agg3, nd, W5, b5.reshape(1, D))
    return h[:N], hc[:N]
